# baseline (speedup 1.0000x reference)
import functools

import jax
import jax.numpy as jnp
from jax import lax
from jax.experimental import pallas as pl
from jax.experimental.pallas import tpu as pltpu
from jax.experimental.pallas import tpu_sc as plsc

BATCH = 16384
HIST = 200
WORDS = HIST // 4
VPAD = 128
NUM_WORKERS = 32
ROWS_PER_W = BATCH // NUM_WORKERS
WORDS_PER_W = ROWS_PER_W * WORDS
NBUF = 3


def _hist_body(x_hbm, counts_hbm, x_v, wide_v, *bufs):
    nc = 2
    wid = lax.axis_index("s") * nc + lax.axis_index("c")

    pltpu.sync_copy(x_hbm.at[pl.ds(wid * WORDS_PER_W, WORDS_PER_W)], x_v)

    zeros = jnp.zeros((16,), jnp.float32)
    iota16 = lax.iota(jnp.int32, 16)
    ones = jnp.ones((16,), jnp.float32)

    @plsc.parallel_loop(0, ROWS_PER_W)
    def _zero(i):
        for b in bufs:
            b[i, pl.ds(0, 16)] = zeros
            b[i, pl.ds(8, 16)] = zeros
        for c in range(2, 8):
            wide_v[i, pl.ds(c * 16, 16)] = zeros

    @plsc.parallel_loop(0, ROWS_PER_W // 16)
    def _groups(jg):
        rows = iota16 + jg * 16
        rows_w = rows * WORDS

        @plsc.parallel_loop(0, WORDS, unroll=2)
        def _hist_l(lw):
            w = plsc.load_gather(x_v, [rows_w + lw])
            ids0 = w & 0xFF
            ids1 = lax.shift_right_logical(w, 8) & 0xFF
            ids2 = lax.shift_right_logical(w, 16) & 0xFF
            ids3 = lax.shift_right_logical(w, 24)
            plsc.addupdate_scatter(bufs[0], [rows, ids0], ones)
            plsc.addupdate_scatter(bufs[1], [rows, ids1], ones)
            plsc.addupdate_scatter(bufs[2], [rows, ids2], ones)
            plsc.addupdate_scatter(bufs[0], [rows, ids3], ones)

    @plsc.parallel_loop(0, ROWS_PER_W)
    def _merge(i):
        lo = bufs[0][i, pl.ds(0, 16)]
        hi = bufs[0][i, pl.ds(8, 16)]
        for b in bufs[1:]:
            lo = lo + b[i, pl.ds(0, 16)]
            hi = hi + b[i, pl.ds(8, 16)]
        wide_v[i, pl.ds(0, 16)] = lo
        wide_v[i, pl.ds(8, 16)] = hi

    pltpu.sync_copy(wide_v, counts_hbm.at[pl.ds(wid * ROWS_PER_W, ROWS_PER_W), :])


@functools.cache
def _make_hist():
    return pl.kernel(
        _hist_body,
        mesh=plsc.VectorSubcoreMesh(core_axis_name="c", subcore_axis_name="s"),
        out_type=jax.ShapeDtypeStruct((BATCH, VPAD), jnp.float32),
        scratch_types=[
            pltpu.VMEM((WORDS_PER_W,), jnp.int32),
            pltpu.VMEM((ROWS_PER_W, VPAD), jnp.float32),
        ]
        + [pltpu.VMEM((ROWS_PER_W, 24), jnp.float32) for _ in range(NBUF)],
        compiler_params=pltpu.CompilerParams(
            use_tc_tiling_on_sc=False,
            needs_layout_passes=False,
        ),
    )


def _matmul_body(c_ref, t_ref, o_ref):
    o_ref[:, :] = jnp.tanh(
        jnp.dot(
            c_ref[:, :],
            t_ref[:, :],
            preferred_element_type=jnp.float32,
            precision=lax.Precision.HIGHEST,
        )
    )


def _matmul_tanh(counts, table128):
    blk = 2048
    return pl.pallas_call(
        _matmul_body,
        grid=(BATCH // blk,),
        in_specs=[
            pl.BlockSpec((blk, VPAD), lambda i: (i, 0)),
            pl.BlockSpec((VPAD, 128), lambda i: (0, 0)),
        ],
        out_specs=pl.BlockSpec((blk, 128), lambda i: (i, 0)),
        out_shape=jax.ShapeDtypeStruct((BATCH, 128), jnp.float32),
    )(counts, table128)


def kernel(x, table):
    x_pk = lax.bitcast_convert_type(
        x.astype(jnp.int8).reshape(BATCH, WORDS, 4), jnp.int32
    ).reshape(-1)
    counts = _make_hist()(x_pk)
    table128 = jnp.concatenate([table, jnp.zeros((105, 128), table.dtype)], axis=0)
    return _matmul_tanh(counts, table128)
